# baseline (speedup 1.0000x reference)
import jax
import jax.numpy as jnp
from jax.experimental import pallas as pl


_BLOCK_S = 256


def _body(x_ref, emb_ref, o_ref):
    D = emb_ref.shape[-1]
    B = x_ref.shape[-1] // D
    emb = emb_ref[...]
    for b in range(B):
        o_ref[:, b * D:(b + 1) * D] = x_ref[:, b * D:(b + 1) * D] + emb


def kernel(x, pos_emb_table):
    S, B, D = x.shape
    x2 = x.reshape(S, B * D)
    grid = (S // _BLOCK_S,)
    out = pl.pallas_call(
        _body,
        grid=grid,
        in_specs=[
            pl.BlockSpec((_BLOCK_S, B * D), lambda i: (i, 0)),
            pl.BlockSpec((_BLOCK_S, D), lambda i: (i, 0)),
        ],
        out_specs=pl.BlockSpec((_BLOCK_S, B * D), lambda i: (i, 0)),
        out_shape=jax.ShapeDtypeStruct((S, B * D), x.dtype),
    )(x2, pos_emb_table[:S])
    return out.reshape(S, B, D)


# speedup vs baseline: 4.2142x; 4.2142x over previous
import jax
import jax.numpy as jnp
from jax.experimental import pallas as pl


_BLOCK_S = 512


def _body(x_ref, emb_ref, o_ref):
    o_ref[...] = x_ref[...] + emb_ref[...][:, None, :]


def kernel(x, pos_emb_table):
    S, B, D = x.shape
    grid = (S // _BLOCK_S,)
    return pl.pallas_call(
        _body,
        grid=grid,
        in_specs=[
            pl.BlockSpec((_BLOCK_S, B, D), lambda i: (i, 0, 0)),
            pl.BlockSpec((_BLOCK_S, D), lambda i: (i, 0)),
        ],
        out_specs=pl.BlockSpec((_BLOCK_S, B, D), lambda i: (i, 0, 0)),
        out_shape=jax.ShapeDtypeStruct((S, B, D), x.dtype),
    )(x, pos_emb_table[:S])
